# baseline (speedup 1.0000x reference)
import jax
import jax.numpy as jnp
from jax import lax
from jax.experimental import pallas as pl
from jax.experimental.pallas import tpu as pltpu
from jax.experimental.pallas import tpu_sc as plsc

_B = 16384
_D = 64
_W = 128
_MARGIN = 1.0
_NC = 2
_NS = 16
_NW = _NC * _NS
_L = 16
_BPW = _B // _NW
_CH = 128
_NCH = _BPW // _CH


def _hsum16(v, lanes):
    dnums = lax.GatherDimensionNumbers(offset_dims=(),
                                       collapsed_slice_dims=(0,),
                                       start_index_map=(0,))
    for s in (8, 4, 2, 1):
        idx = jnp.bitwise_xor(lanes, jnp.int32(s)).reshape(_L, 1)
        p = lax.gather(v, idx, dnums, (1,),
                       mode=lax.GatherScatterMode.PROMISE_IN_BOUNDS)
        v = v + p
    return v


def _sqrt16(x):
    x = jnp.maximum(x, jnp.float32(1e-30))
    i = lax.bitcast_convert_type(x, jnp.int32)
    i = jnp.int32(0x5F3759DF) - lax.shift_right_arithmetic(i, jnp.int32(1))
    r = lax.bitcast_convert_type(i, jnp.float32)
    half = jnp.float32(0.5) * x
    for _ in range(3):
        r = r * (jnp.float32(1.5) - half * r * r)
    return x * r


def _tec_body(ent_hbm, rel_hbm, ph_hbm, pr_hbm, pt_hbm, nh_hbm, nr_hbm,
              nt_hbm, out_hbm, i_h, i_r, i_t, s_h, s_r, s_t,
              buf_h, buf_r, buf_t, sp, sn, sem):
    wid = lax.axis_index("s") * _NC + lax.axis_index("c")
    base = wid * _BPW
    lanes = lax.broadcasted_iota(jnp.int32, (_L,), 0)

    def side(ih_hbm, ir_hbm, it_hbm, sq_out):
        pltpu.sync_copy(ih_hbm.at[pl.ds(base, _BPW)], i_h)
        pltpu.sync_copy(ir_hbm.at[pl.ds(base, _BPW)], i_r)
        pltpu.sync_copy(it_hbm.at[pl.ds(base, _BPW)], i_t)

        def shift_body(v, src, dst):
            vsl = pl.ds(v * _L, _L)
            dst[vsl] = lax.shift_right_logical(src[vsl], jnp.int32(1))
            return None

        def shift_all(v, carry):
            shift_body(v, i_h, s_h)
            shift_body(v, i_r, s_r)
            shift_body(v, i_t, s_t)
            return carry

        lax.fori_loop(0, _BPW // _L, shift_all, jnp.int32(0))

        for c in range(_NCH):
            sl = pl.ds(c * _CH, _CH)
            cps = [
                pltpu.async_copy(ent_hbm.at[s_h.at[sl]], buf_h, sem),
                pltpu.async_copy(rel_hbm.at[s_r.at[sl]], buf_r, sem),
                pltpu.async_copy(ent_hbm.at[s_t.at[sl]], buf_t, sem),
            ]
            for cp in cps:
                cp.wait()

            def group_body(g, carry):
                gbase = c * _CH + g * _L
                gsl = pl.ds(gbase, _L)
                offh = (i_h[gsl] & jnp.int32(1)) * jnp.int32(_D)
                offr = (i_r[gsl] & jnp.int32(1)) * jnp.int32(_D)
                offt = (i_t[gsl] & jnp.int32(1)) * jnp.int32(_D)

                vec = jnp.zeros((_L,), jnp.float32)
                for i in range(_L):
                    row = g * _L + i
                    oh, orr, ot = offh[i], offr[i], offt[i]
                    acc = jnp.zeros((_L,), jnp.float32)
                    for k in range(_D // _L):
                        o = jnp.int32(k * _L)
                        d = ((buf_h[row, pl.ds(oh + o, _L)]
                              + buf_r[row, pl.ds(orr + o, _L)])
                             - buf_t[row, pl.ds(ot + o, _L)])
                        acc = acc + d * d
                    tot = _hsum16(acc, lanes)
                    vec = jnp.where(lanes == i, tot, vec)
                sq_out[pl.ds(gbase, _L)] = vec
                return carry

            lax.fori_loop(0, _CH // _L, group_body, jnp.int32(0))

    side(ph_hbm, pr_hbm, pt_hbm, sp)
    side(nh_hbm, nr_hbm, nt_hbm, sn)

    def loss_body(g, lacc):
        gsl = pl.ds(g * _L, _L)
        spv = _sqrt16(sp[gsl])
        snv = _sqrt16(sn[gsl])
        return lacc + jnp.maximum(jnp.float32(_MARGIN) + spv - snv,
                                  jnp.float32(0.0))

    lossacc = lax.fori_loop(0, _BPW // _L, loss_body,
                            jnp.zeros((_L,), jnp.float32))
    sp[pl.ds(0, _L)] = lossacc
    pltpu.sync_copy(sp.at[pl.ds(0, _L)], out_hbm.at[pl.ds(wid * _L, _L)])


@jax.jit
def _transe_sc(entity_emb2, relation_emb2, ph, pr, pt, nh, nr, nt):
    call = pl.kernel(
        _tec_body,
        out_type=jax.ShapeDtypeStruct((_NW * _L,), jnp.float32),
        mesh=plsc.VectorSubcoreMesh(core_axis_name="c", subcore_axis_name="s",
                                    num_cores=_NC, num_subcores=_NS),
        scratch_types=[
            pltpu.VMEM((_BPW,), jnp.int32),
            pltpu.VMEM((_BPW,), jnp.int32),
            pltpu.VMEM((_BPW,), jnp.int32),
            pltpu.VMEM((_BPW,), jnp.int32),
            pltpu.VMEM((_BPW,), jnp.int32),
            pltpu.VMEM((_BPW,), jnp.int32),
            pltpu.VMEM((_CH, _W), jnp.float32),
            pltpu.VMEM((_CH, _W), jnp.float32),
            pltpu.VMEM((_CH, _W), jnp.float32),
            pltpu.VMEM((_BPW,), jnp.float32),
            pltpu.VMEM((_BPW,), jnp.float32),
            pltpu.SemaphoreType.DMA,
        ],
    )
    return call(entity_emb2, relation_emb2, ph, pr, pt, nh, nr, nt)


def kernel(pos_triples, neg_triples, entity_emb, relation_emb):
    p = pos_triples.astype(jnp.int32)
    n = neg_triples.astype(jnp.int32)
    ent2 = entity_emb.reshape(entity_emb.shape[0] // 2, _W)
    rel2 = relation_emb.reshape(relation_emb.shape[0] // 2, _W)
    partials = _transe_sc(ent2, rel2,
                          p[:, 0], p[:, 1], p[:, 2],
                          n[:, 0], n[:, 1], n[:, 2])
    return jnp.sum(partials) * jnp.float32(1.0 / _B)


# speedup vs baseline: 2.3056x; 2.3056x over previous
import jax
import jax.numpy as jnp
from jax import lax
from jax.experimental import pallas as pl
from jax.experimental.pallas import tpu as pltpu
from jax.experimental.pallas import tpu_sc as plsc

_B = 16384
_D = 64
_MARGIN = 1.0
_NC = 2
_NS = 16
_NW = _NC * _NS
_L = 16
_BPW = _B // _NW
_CH = 128
_NCH = _BPW // _CH


def _hsum16(v, lanes):
    dnums = lax.GatherDimensionNumbers(offset_dims=(),
                                       collapsed_slice_dims=(0,),
                                       start_index_map=(0,))
    for s in (8, 4, 2, 1):
        idx = jnp.bitwise_xor(lanes, jnp.int32(s)).reshape(_L, 1)
        p = lax.gather(v, idx, dnums, (1,),
                       mode=lax.GatherScatterMode.PROMISE_IN_BOUNDS)
        v = v + p
    return v


def _sqrt16(x):
    x = jnp.maximum(x, jnp.float32(1e-30))
    i = lax.bitcast_convert_type(x, jnp.int32)
    i = jnp.int32(0x5F3759DF) - lax.shift_right_arithmetic(i, jnp.int32(1))
    r = lax.bitcast_convert_type(i, jnp.float32)
    half = jnp.float32(0.5) * x
    for _ in range(3):
        r = r * (jnp.float32(1.5) - half * r * r)
    return x * r


def _tec_body(ent_hbm, rel_hbm, ph_hbm, pr_hbm, pt_hbm, nh_hbm, nr_hbm,
              nt_hbm, out_hbm, i_h, i_r, i_t,
              buf_h, buf_r, buf_t, sp, sn, sem):
    wid = lax.axis_index("s") * _NC + lax.axis_index("c")
    base = wid * _BPW
    lanes = lax.broadcasted_iota(jnp.int32, (_L,), 0)

    def side(ih_hbm, ir_hbm, it_hbm, sq_out):
        pltpu.sync_copy(ih_hbm.at[pl.ds(base, _BPW)], i_h)
        pltpu.sync_copy(ir_hbm.at[pl.ds(base, _BPW)], i_r)
        pltpu.sync_copy(it_hbm.at[pl.ds(base, _BPW)], i_t)

        for c in range(_NCH):
            sl = pl.ds(c * _CH, _CH)
            cps = [
                pltpu.async_copy(ent_hbm.at[i_h.at[sl]], buf_h, sem),
                pltpu.async_copy(rel_hbm.at[i_r.at[sl]], buf_r, sem),
                pltpu.async_copy(ent_hbm.at[i_t.at[sl]], buf_t, sem),
            ]
            for cp in cps:
                cp.wait()

            def group_body(g, carry):
                gbase = c * _CH + g * _L
                vec = jnp.zeros((_L,), jnp.float32)
                for i in range(_L):
                    row = g * _L + i
                    acc = jnp.zeros((_L,), jnp.float32)
                    for k in range(_D // _L):
                        o = k * _L
                        d = ((buf_h[row, 0, pl.ds(o, _L)]
                              + buf_r[row, 0, pl.ds(o, _L)])
                             - buf_t[row, 0, pl.ds(o, _L)])
                        acc = acc + d * d
                    tot = _hsum16(acc, lanes)
                    vec = jnp.where(lanes == i, tot, vec)
                sq_out[pl.ds(gbase, _L)] = vec
                return carry

            lax.fori_loop(0, _CH // _L, group_body, jnp.int32(0))

    side(ph_hbm, pr_hbm, pt_hbm, sp)
    side(nh_hbm, nr_hbm, nt_hbm, sn)

    def loss_body(g, lacc):
        gsl = pl.ds(g * _L, _L)
        spv = _sqrt16(sp[gsl])
        snv = _sqrt16(sn[gsl])
        return lacc + jnp.maximum(jnp.float32(_MARGIN) + spv - snv,
                                  jnp.float32(0.0))

    lossacc = lax.fori_loop(0, _BPW // _L, loss_body,
                            jnp.zeros((_L,), jnp.float32))
    sp[pl.ds(0, _L)] = lossacc
    pltpu.sync_copy(sp.at[pl.ds(0, _L)], out_hbm.at[pl.ds(wid * _L, _L)])


@jax.jit
def _transe_sc(entity_emb3, relation_emb3, ph, pr, pt, nh, nr, nt):
    call = pl.kernel(
        _tec_body,
        out_type=jax.ShapeDtypeStruct((_NW * _L,), jnp.float32),
        mesh=plsc.VectorSubcoreMesh(core_axis_name="c", subcore_axis_name="s",
                                    num_cores=_NC, num_subcores=_NS),
        scratch_types=[
            pltpu.VMEM((_BPW,), jnp.int32),
            pltpu.VMEM((_BPW,), jnp.int32),
            pltpu.VMEM((_BPW,), jnp.int32),
            pltpu.VMEM((_CH, 1, _D), jnp.float32),
            pltpu.VMEM((_CH, 1, _D), jnp.float32),
            pltpu.VMEM((_CH, 1, _D), jnp.float32),
            pltpu.VMEM((_BPW,), jnp.float32),
            pltpu.VMEM((_BPW,), jnp.float32),
            pltpu.SemaphoreType.DMA,
        ],
    )
    return call(entity_emb3, relation_emb3, ph, pr, pt, nh, nr, nt)


def kernel(pos_triples, neg_triples, entity_emb, relation_emb):
    p = pos_triples.astype(jnp.int32)
    n = neg_triples.astype(jnp.int32)
    ent3 = entity_emb.reshape(entity_emb.shape[0], 1, _D)
    rel3 = relation_emb.reshape(relation_emb.shape[0], 1, _D)
    partials = _transe_sc(ent3, rel3,
                          p[:, 0], p[:, 1], p[:, 2],
                          n[:, 0], n[:, 1], n[:, 2])
    return jnp.sum(partials) * jnp.float32(1.0 / _B)


# speedup vs baseline: 2.3726x; 1.0291x over previous
import jax
import jax.numpy as jnp
from jax import lax
from jax.experimental import pallas as pl
from jax.experimental.pallas import tpu as pltpu
from jax.experimental.pallas import tpu_sc as plsc

_B = 16384
_D = 64
_MARGIN = 1.0
_NC = 2
_NS = 16
_NW = _NC * _NS
_L = 16
_BPW = _B // _NW
_CH = 128
_NCH = _BPW // _CH


def _hsum16(v, lanes):
    dnums = lax.GatherDimensionNumbers(offset_dims=(),
                                       collapsed_slice_dims=(0,),
                                       start_index_map=(0,))
    for s in (8, 4, 2, 1):
        idx = jnp.bitwise_xor(lanes, jnp.int32(s)).reshape(_L, 1)
        p = lax.gather(v, idx, dnums, (1,),
                       mode=lax.GatherScatterMode.PROMISE_IN_BOUNDS)
        v = v + p
    return v


def _sqrt16(x):
    x = jnp.maximum(x, jnp.float32(1e-30))
    i = lax.bitcast_convert_type(x, jnp.int32)
    i = jnp.int32(0x5F3759DF) - lax.shift_right_arithmetic(i, jnp.int32(1))
    r = lax.bitcast_convert_type(i, jnp.float32)
    half = jnp.float32(0.5) * x
    for _ in range(3):
        r = r * (jnp.float32(1.5) - half * r * r)
    return x * r


def _tec_body(ent_hbm, rel_hbm, ph_hbm, pr_hbm, pt_hbm, nh_hbm, nr_hbm,
              nt_hbm, out_hbm, pi_h, pi_r, pi_t, ni_h, ni_r, ni_t,
              bh0, br0, bt0, bh1, br1, bt1, sp, sn, sem0, sem1):
    wid = lax.axis_index("s") * _NC + lax.axis_index("c")
    base = wid * _BPW
    lanes = lax.broadcasted_iota(jnp.int32, (_L,), 0)

    pltpu.sync_copy(ph_hbm.at[pl.ds(base, _BPW)], pi_h)
    pltpu.sync_copy(pr_hbm.at[pl.ds(base, _BPW)], pi_r)
    pltpu.sync_copy(pt_hbm.at[pl.ds(base, _BPW)], pi_t)
    pltpu.sync_copy(nh_hbm.at[pl.ds(base, _BPW)], ni_h)
    pltpu.sync_copy(nr_hbm.at[pl.ds(base, _BPW)], ni_r)
    pltpu.sync_copy(nt_hbm.at[pl.ds(base, _BPW)], ni_t)

    slots = ((bh0, br0, bt0, sem0), (bh1, br1, bt1, sem1))
    sides = ((pi_h, pi_r, pi_t, sp), (ni_h, ni_r, ni_t, sn))

    def issue(c):
        i_h, i_r, i_t, _ = sides[c // _NCH]
        bh, br, bt, sem = slots[c % 2]
        sl = pl.ds((c % _NCH) * _CH, _CH)
        return [
            pltpu.async_copy(ent_hbm.at[i_h.at[sl]], bh, sem),
            pltpu.async_copy(rel_hbm.at[i_r.at[sl]], br, sem),
            pltpu.async_copy(ent_hbm.at[i_t.at[sl]], bt, sem),
        ]

    pending = issue(0)
    for c in range(2 * _NCH):
        nxt = issue(c + 1) if c + 1 < 2 * _NCH else None
        for cp in pending:
            cp.wait()
        bh, br, bt, _ = slots[c % 2]
        sq_out = sides[c // _NCH][3]
        cbase = (c % _NCH) * _CH

        def group_body(g, carry):
            vec = jnp.zeros((_L,), jnp.float32)
            for i in range(_L):
                row = g * _L + i
                acc = jnp.zeros((_L,), jnp.float32)
                for k in range(_D // _L):
                    o = k * _L
                    d = ((bh[row, 0, pl.ds(o, _L)]
                          + br[row, 0, pl.ds(o, _L)])
                         - bt[row, 0, pl.ds(o, _L)])
                    acc = acc + d * d
                tot = _hsum16(acc, lanes)
                vec = jnp.where(lanes == i, tot, vec)
            sq_out[pl.ds(cbase + g * _L, _L)] = vec
            return carry

        lax.fori_loop(0, _CH // _L, group_body, jnp.int32(0))
        pending = nxt

    def loss_body(g, lacc):
        gsl = pl.ds(g * _L, _L)
        spv = _sqrt16(sp[gsl])
        snv = _sqrt16(sn[gsl])
        return lacc + jnp.maximum(jnp.float32(_MARGIN) + spv - snv,
                                  jnp.float32(0.0))

    lossacc = lax.fori_loop(0, _BPW // _L, loss_body,
                            jnp.zeros((_L,), jnp.float32))
    sp[pl.ds(0, _L)] = lossacc
    pltpu.sync_copy(sp.at[pl.ds(0, _L)], out_hbm.at[pl.ds(wid * _L, _L)])


@jax.jit
def _transe_sc(entity_emb3, relation_emb3, ph, pr, pt, nh, nr, nt):
    call = pl.kernel(
        _tec_body,
        out_type=jax.ShapeDtypeStruct((_NW * _L,), jnp.float32),
        mesh=plsc.VectorSubcoreMesh(core_axis_name="c", subcore_axis_name="s",
                                    num_cores=_NC, num_subcores=_NS),
        scratch_types=[
            pltpu.VMEM((_BPW,), jnp.int32),
            pltpu.VMEM((_BPW,), jnp.int32),
            pltpu.VMEM((_BPW,), jnp.int32),
            pltpu.VMEM((_BPW,), jnp.int32),
            pltpu.VMEM((_BPW,), jnp.int32),
            pltpu.VMEM((_BPW,), jnp.int32),
            pltpu.VMEM((_CH, 1, _D), jnp.float32),
            pltpu.VMEM((_CH, 1, _D), jnp.float32),
            pltpu.VMEM((_CH, 1, _D), jnp.float32),
            pltpu.VMEM((_CH, 1, _D), jnp.float32),
            pltpu.VMEM((_CH, 1, _D), jnp.float32),
            pltpu.VMEM((_CH, 1, _D), jnp.float32),
            pltpu.VMEM((_BPW,), jnp.float32),
            pltpu.VMEM((_BPW,), jnp.float32),
            pltpu.SemaphoreType.DMA,
            pltpu.SemaphoreType.DMA,
        ],
    )
    return call(entity_emb3, relation_emb3, ph, pr, pt, nh, nr, nt)


def kernel(pos_triples, neg_triples, entity_emb, relation_emb):
    p = pos_triples.astype(jnp.int32)
    n = neg_triples.astype(jnp.int32)
    ent3 = entity_emb.reshape(entity_emb.shape[0], 1, _D)
    rel3 = relation_emb.reshape(relation_emb.shape[0], 1, _D)
    partials = _transe_sc(ent3, rel3,
                          p[:, 0], p[:, 1], p[:, 2],
                          n[:, 0], n[:, 1], n[:, 2])
    return jnp.sum(partials) * jnp.float32(1.0 / _B)
